# baseline (speedup 1.0000x reference)
import jax
import jax.numpy as jnp
from jax import lax
from jax.experimental import pallas as pl
from jax.experimental.pallas import tpu as pltpu
from jax.experimental.pallas import tpu_sc as plsc

N = 10000
E = 160000
D = 256
PAD = 10240
NC, NS, L = 2, 16, 16
NW = NC * NS
EPS = E // 16
GROUPS = EPS // L
KPAD = 10240
DUMMY = PAD - 1


def _build_body(edge_hbm, b_hbm, s_v, d_v, key_v, ones_v, sem):
    c = lax.axis_index("c")
    s = lax.axis_index("s")
    wid = s * NC + c
    m = lax.rem(wid, 16)
    flip = lax.div(wid, 16)
    base = m * EPS
    pltpu.sync_copy(edge_hbm.at[pl.ds(base, EPS)], s_v)
    pltpu.sync_copy(edge_hbm.at[pl.ds(E + base, EPS)], d_v)

    one = jnp.full((L,), 1.0, jnp.float32)
    dummy = jnp.full((L,), DUMMY, jnp.int32)

    def ob(g, carry):
        ones_v[pl.ds(g * L, L)] = one
        return carry

    lax.fori_loop(0, KPAD // L, ob, 0)

    for g in range(GROUPS, KPAD // L):
        key_v[pl.ds(g * L, L)] = dummy

    def kb(g, carry):
        sv = s_v[pl.ds(g * L, L)]
        dv = d_v[pl.ds(g * L, L)]
        kv = dv * PAD + sv + flip * ((sv - dv) * (PAD - 1))
        key_v[pl.ds(g * L, L)] = kv
        return carry

    lax.fori_loop(0, GROUPS, kb, 0)

    pltpu.async_copy(ones_v, b_hbm.at[key_v], sem).wait()


_scatter_ones = pl.kernel(
    _build_body,
    out_type=(),
    mesh=plsc.VectorSubcoreMesh(core_axis_name="c", subcore_axis_name="s"),
    scratch_types=[
        pltpu.VMEM((EPS,), jnp.int32),
        pltpu.VMEM((EPS,), jnp.int32),
        pltpu.VMEM((KPAD,), jnp.int32),
        pltpu.VMEM((KPAD,), jnp.float32),
        pltpu.SemaphoreType.DMA,
    ],
)


def _mm1_body(b_blk, hb_full, h_blk, o_blk, bb_blk):
    bb = jnp.minimum(b_blk[...], 1.0).astype(jnp.bfloat16)
    bb_blk[...] = bb
    o_blk[...] = h_blk[...] + jnp.dot(
        bb, hb_full[...], preferred_element_type=jnp.float32
    )


def _layer1(bmat, hb, h):
    return pl.pallas_call(
        _mm1_body,
        grid=(PAD // 256,),
        in_specs=[
            pl.BlockSpec((256, PAD), lambda i: (i, 0)),
            pl.BlockSpec((PAD, D), lambda i: (0, 0)),
            pl.BlockSpec((256, D), lambda i: (i, 0)),
        ],
        out_specs=[
            pl.BlockSpec((256, D), lambda i: (i, 0)),
            pl.BlockSpec((256, PAD), lambda i: (i, 0)),
        ],
        out_shape=[
            jax.ShapeDtypeStruct((PAD, D), jnp.float32),
            jax.ShapeDtypeStruct((PAD, PAD), jnp.bfloat16),
        ],
    )(bmat, hb, h)


def _mm_body(bb_blk, hb_full, h_blk, o_blk):
    o_blk[...] = h_blk[...] + jnp.dot(
        bb_blk[...], hb_full[...], preferred_element_type=jnp.float32
    )


def _layer(bb, hb, h):
    return pl.pallas_call(
        _mm_body,
        grid=(PAD // 256,),
        in_specs=[
            pl.BlockSpec((256, PAD), lambda i: (i, 0)),
            pl.BlockSpec((PAD, D), lambda i: (0, 0)),
            pl.BlockSpec((256, D), lambda i: (i, 0)),
        ],
        out_specs=pl.BlockSpec((256, D), lambda i: (i, 0)),
        out_shape=jax.ShapeDtypeStruct((PAD, D), jnp.float32),
    )(bb, hb, h)


def kernel(x, edge_index):
    bref = jax.new_ref(jnp.zeros((PAD * PAD,), jnp.float32))
    _scatter_ones(edge_index.reshape(-1), bref)
    bmat = bref[...].reshape(PAD, PAD)
    h = jnp.zeros((PAD, D), jnp.float32).at[:N].set(x)
    h, bb = _layer1(bmat, h.astype(jnp.bfloat16), h)
    for _ in range(2):
        h = _layer(bb, h.astype(jnp.bfloat16), h)
    return h[:N]


# speedup vs baseline: 1.3639x; 1.3639x over previous
import jax
import jax.numpy as jnp
from jax import lax
from jax.experimental import pallas as pl
from jax.experimental.pallas import tpu as pltpu
from jax.experimental.pallas import tpu_sc as plsc

N = 10000
E = 160000
D = 256
PAD = 10240
L = 16
EPS = E // 16
GROUPS = EPS // L
KN = 20480
KNG = KN // L
EC = 2000
OUT_KEY = PAD * PAD
HALFK = (PAD // 2) * PAD

SR = 128
SPW = SR * PAD
PASSES = (PAD // 2) // SR
SHARE = SPW // 16
ZN = 2048
CH = 128
CROWS = (KN + CH - 1) // CH


def _build_body(edge_hbm, b_hbm, s_c, d_c, key_v, chunk_v,
                zero_v, ones_v, sp, sem):
    c = lax.axis_index("c")
    sidx = lax.axis_index("s")
    base = sidx * EPS

    zero32f = jnp.zeros((L,), jnp.float32)
    onef = jnp.full((L,), 1.0, jnp.float32)

    def zb(g, carry):
        zero_v[pl.ds(g * L, L)] = zero32f
        return carry

    lax.fori_loop(0, ZN // L, zb, 0)
    for g in range(CH // L):
        ones_v[pl.ds(g * L, L)] = onef

    big = jnp.full((L,), OUT_KEY, jnp.int32)

    def bigb(g, carry):
        key_v[pl.ds(g * L, L)] = big
        return carry

    lax.fori_loop(0, KNG, bigb, 0)

    io16 = jnp.arange(L, dtype=jnp.int32)
    one_i = jnp.int32(1)
    zero_i = jnp.int32(0)

    def kb(g, carry):
        cnt0, cnt1 = carry
        sv = s_c[pl.ds(g * L, L)]
        dv = d_c[pl.ds(g * L, L)]
        for kv in (dv * PAD + sv, sv * PAD + dv):
            m0 = kv < HALFK
            t = jnp.where(m0, cnt0 * L, (KNG - 1 - cnt1) * L) + io16
            plsc.store_scatter(key_v, [t], kv)
            cnt0 = cnt0 + jnp.where(m0, one_i, zero_i)
            cnt1 = cnt1 + jnp.where(m0, zero_i, one_i)
        return cnt0, cnt1

    zv = jnp.zeros((L,), jnp.int32)
    carry = (zv, zv)
    for e in range(EPS // EC):
        pltpu.sync_copy(edge_hbm.at[pl.ds(base + e * EC, EC)], s_c)
        pltpu.sync_copy(edge_hbm.at[pl.ds(E + base + e * EC, EC)], d_c)
        carry = lax.fori_loop(0, EC // L, kb, carry)
    cnt0, cnt1 = carry

    def _vmax(v):
        m = v[0]
        for j in range(1, L):
            m = jnp.maximum(m, v[j])
        return m

    g_lo = jnp.where(c == 0, 0, KNG - _vmax(cnt1))
    g_hi = jnp.where(c == 0, _vmax(cnt0), KNG)

    garbv = jnp.full((L,), SPW, jnp.int32)

    def garb_rows(r, carry):
        for l in range(CH // L):
            chunk_v[r, pl.ds(l * L, L)] = garbv
        return carry

    lax.fori_loop(0, CROWS, garb_rows, 0)
    TRASH_ROW = jnp.int32(CROWS)

    def pass_body(p, carry):
        lo = (c * (PAD // 2) + p * SR) * PAD
        hi = lo + SPW

        for z in range(SHARE // ZN):
            pltpu.sync_copy(zero_v,
                            sp.at[pl.ds(sidx * SHARE + z * ZN, ZN)])
        plsc.subcore_barrier()

        def fb(g, cnt_v):
            kv = key_v[pl.ds(g * L, L)]
            m = (kv >= lo) & (kv < hi)
            slot = cnt_v * L + io16
            row = jnp.right_shift(slot, 7)
            col = jnp.bitwise_and(slot, jnp.int32(CH - 1))
            t_row = jnp.where(m, row, TRASH_ROW)
            plsc.store_scatter(chunk_v, [t_row, col], kv - lo)
            return cnt_v + jnp.where(m, one_i, zero_i)

        cnt_v = lax.fori_loop(g_lo, g_hi, fb, zv)
        cmax = _vmax(cnt_v)
        nch = lax.div(cmax * L + (CH - 1), CH)

        def cb(r, carry):
            pltpu.async_copy(ones_v, sp.at[chunk_v.at[r]], sem,
                             add=True).wait()
            return carry

        lax.fori_loop(0, nch, cb, 0)
        plsc.subcore_barrier()

        pltpu.sync_copy(sp.at[pl.ds(sidx * SHARE, SHARE)],
                        b_hbm.at[pl.ds(lo + sidx * SHARE, SHARE)])
        lax.fori_loop(0, nch, garb_rows, 0)
        return carry

    lax.fori_loop(0, PASSES, pass_body, 0)


_build_b = pl.kernel(
    _build_body,
    out_type=(),
    mesh=plsc.VectorSubcoreMesh(core_axis_name="c", subcore_axis_name="s"),
    compiler_params=pltpu.CompilerParams(needs_layout_passes=False),
    scratch_types=[
        pltpu.VMEM((EC,), jnp.int32),
        pltpu.VMEM((EC,), jnp.int32),
        pltpu.VMEM((KN,), jnp.int32),
        pltpu.VMEM((CROWS + 1, CH), jnp.int32),
        pltpu.VMEM((ZN,), jnp.float32),
        pltpu.VMEM((CH,), jnp.float32),
        pltpu.VMEM_SHARED((SPW + L,), jnp.float32),
        pltpu.SemaphoreType.DMA,
    ],
)


def _mm1_body(b_blk, hb_full, h_blk, o_blk, bb_blk):
    bb = jnp.minimum(b_blk[...], 1.0).astype(jnp.bfloat16)
    bb_blk[...] = bb
    o_blk[...] = h_blk[...] + jnp.dot(
        bb, hb_full[...], preferred_element_type=jnp.float32
    )


def _layer1(bmat, hb, h):
    return pl.pallas_call(
        _mm1_body,
        grid=(PAD // 256,),
        in_specs=[
            pl.BlockSpec((256, PAD), lambda i: (i, 0)),
            pl.BlockSpec((PAD, D), lambda i: (0, 0)),
            pl.BlockSpec((256, D), lambda i: (i, 0)),
        ],
        out_specs=[
            pl.BlockSpec((256, D), lambda i: (i, 0)),
            pl.BlockSpec((256, PAD), lambda i: (i, 0)),
        ],
        out_shape=[
            jax.ShapeDtypeStruct((PAD, D), jnp.float32),
            jax.ShapeDtypeStruct((PAD, PAD), jnp.bfloat16),
        ],
    )(bmat, hb, h)


def _mm_body(bb_blk, hb_full, h_blk, o_blk):
    o_blk[...] = h_blk[...] + jnp.dot(
        bb_blk[...], hb_full[...], preferred_element_type=jnp.float32
    )


def _layer(bb, hb, h):
    return pl.pallas_call(
        _mm_body,
        grid=(PAD // 256,),
        in_specs=[
            pl.BlockSpec((256, PAD), lambda i: (i, 0)),
            pl.BlockSpec((PAD, D), lambda i: (0, 0)),
            pl.BlockSpec((256, D), lambda i: (i, 0)),
        ],
        out_specs=pl.BlockSpec((256, D), lambda i: (i, 0)),
        out_shape=jax.ShapeDtypeStruct((PAD, D), jnp.float32),
    )(bb, hb, h)


def kernel(x, edge_index):
    bref = jax.new_ref(jnp.empty((PAD * PAD,), jnp.float32))
    _build_b(edge_index.reshape(-1), bref)
    bmat = bref[...].reshape(PAD, PAD)
    h = jnp.zeros((PAD, D), jnp.float32).at[:N].set(x)
    h, bb = _layer1(bmat, h.astype(jnp.bfloat16), h)
    for _ in range(2):
        h = _layer(bb, h.astype(jnp.bfloat16), h)
    return h[:N]


# speedup vs baseline: 1.3815x; 1.0129x over previous
import jax
import jax.numpy as jnp
from jax import lax
from jax.experimental import pallas as pl
from jax.experimental.pallas import tpu as pltpu
from jax.experimental.pallas import tpu_sc as plsc

N = 10000
E = 160000
D = 256
PAD = 10240
L = 16
EPS = E // 16
GROUPS = EPS // L
KN = 20480
KNG = KN // L
EC = 2000
OUT_KEY = PAD * PAD
HALFK = (PAD // 2) * PAD

SR = 128
SPW = SR * PAD
PASSES = (PAD // 2) // SR
SHARE = SPW // 16
ZN = 2048
CH = 128
CROWS = (KN + CH - 1) // CH


def _build_body(edge_hbm, b_hbm, s_c, d_c, key_v, chunk_v,
                zero_v, ones_v, sp, sem):
    c = lax.axis_index("c")
    sidx = lax.axis_index("s")
    base = sidx * EPS

    zero32f = jnp.zeros((L,), jnp.float32)
    onef = jnp.full((L,), 1.0, jnp.float32)

    def zb(g, carry):
        zero_v[pl.ds(g * L, L)] = zero32f
        return carry

    lax.fori_loop(0, ZN // L, zb, 0)
    for g in range(CH // L):
        ones_v[pl.ds(g * L, L)] = onef

    big = jnp.full((L,), OUT_KEY, jnp.int32)

    def bigb(g, carry):
        key_v[pl.ds(g * L, L)] = big
        return carry

    lax.fori_loop(0, KNG, bigb, 0)

    io16 = jnp.arange(L, dtype=jnp.int32)
    one_i = jnp.int32(1)
    zero_i = jnp.int32(0)

    def kb(g, carry):
        cnt0, cnt1 = carry
        sv = s_c[pl.ds(g * L, L)]
        dv = d_c[pl.ds(g * L, L)]
        for kv in (dv * PAD + sv, sv * PAD + dv):
            m0 = kv < HALFK
            t = jnp.where(m0, cnt0 * L, (KNG - 1 - cnt1) * L) + io16
            plsc.store_scatter(key_v, [t], kv)
            cnt0 = cnt0 + jnp.where(m0, one_i, zero_i)
            cnt1 = cnt1 + jnp.where(m0, zero_i, one_i)
        return cnt0, cnt1

    zv = jnp.zeros((L,), jnp.int32)
    carry = (zv, zv)
    for e in range(EPS // EC):
        pltpu.sync_copy(edge_hbm.at[pl.ds(base + e * EC, EC)], s_c)
        pltpu.sync_copy(edge_hbm.at[pl.ds(E + base + e * EC, EC)], d_c)
        carry = lax.fori_loop(0, EC // L, kb, carry)
    cnt0, cnt1 = carry

    def _vmax(v):
        m = v[0]
        for j in range(1, L):
            m = jnp.maximum(m, v[j])
        return m

    g_lo = jnp.where(c == 0, 0, KNG - _vmax(cnt1))
    g_hi = jnp.where(c == 0, _vmax(cnt0), KNG)

    garbv = jnp.full((L,), SPW, jnp.int32)

    def garb_rows(r, carry):
        for l in range(CH // L):
            chunk_v[r, pl.ds(l * L, L)] = garbv
        return carry

    lax.fori_loop(0, CROWS, garb_rows, 0)
    TRASH_ROW = jnp.int32(CROWS)

    def pass_body(p, carry):
        lo = (c * (PAD // 2) + p * SR) * PAD
        hi = lo + SPW

        for z in range(SHARE // ZN):
            pltpu.sync_copy(zero_v,
                            sp.at[pl.ds(sidx * SHARE + z * ZN, ZN)])
        plsc.subcore_barrier()

        def fb(g, cnt_v):
            kv = key_v[pl.ds(g * L, L)]
            m = (kv >= lo) & (kv < hi)
            slot = cnt_v * L + io16
            row = jnp.right_shift(slot, 7)
            col = jnp.bitwise_and(slot, jnp.int32(CH - 1))
            t_row = jnp.where(m, row, TRASH_ROW)
            plsc.store_scatter(chunk_v, [t_row, col], kv - lo)
            return cnt_v + jnp.where(m, one_i, zero_i)

        cnt_v = lax.fori_loop(g_lo, g_hi, fb, zv)
        cmax = _vmax(cnt_v)
        nch = lax.div(cmax * L + (CH - 1), CH)

        def cb(r, carry):
            pltpu.async_copy(ones_v, sp.at[chunk_v.at[r]], sem,
                             add=True).wait()
            return carry

        lax.fori_loop(0, nch, cb, 0)
        plsc.subcore_barrier()

        pltpu.sync_copy(sp.at[pl.ds(sidx * SHARE, SHARE)],
                        b_hbm.at[pl.ds(lo + sidx * SHARE, SHARE)])
        lax.fori_loop(0, nch, garb_rows, 0)
        return carry

    lax.fori_loop(0, PASSES, pass_body, 0)


_build_b = pl.kernel(
    _build_body,
    out_type=(),
    mesh=plsc.VectorSubcoreMesh(core_axis_name="c", subcore_axis_name="s"),
    compiler_params=pltpu.CompilerParams(needs_layout_passes=False),
    scratch_types=[
        pltpu.VMEM((EC,), jnp.int32),
        pltpu.VMEM((EC,), jnp.int32),
        pltpu.VMEM((KN,), jnp.int32),
        pltpu.VMEM((CROWS + 1, CH), jnp.int32),
        pltpu.VMEM((ZN,), jnp.float32),
        pltpu.VMEM((CH,), jnp.float32),
        pltpu.VMEM_SHARED((SPW + L,), jnp.float32),
        pltpu.SemaphoreType.DMA,
    ],
)


def _mm1_body(b_blk, hb_full, h_blk, o_blk, bb_blk):
    bb = jnp.minimum(b_blk[...], 1.0).astype(jnp.bfloat16)
    bb_blk[...] = bb
    o_blk[...] = h_blk[...] + jnp.dot(
        bb, hb_full[...], preferred_element_type=jnp.float32
    )


def _layer1(bmat, hb, h):
    return pl.pallas_call(
        _mm1_body,
        grid=(PAD // 256,),
        in_specs=[
            pl.BlockSpec((256, PAD), lambda i: (i, 0)),
            pl.BlockSpec((PAD, D), lambda i: (0, 0)),
            pl.BlockSpec((256, D), lambda i: (i, 0)),
        ],
        out_specs=[
            pl.BlockSpec((256, D), lambda i: (i, 0)),
            pl.BlockSpec((256, PAD), lambda i: (i, 0)),
        ],
        out_shape=[
            jax.ShapeDtypeStruct((PAD, D), jnp.float32),
            jax.ShapeDtypeStruct((PAD, PAD), jnp.bfloat16),
        ],
    )(bmat, hb, h)


def _mm23_body(bb_blk, h1_full, hout_blk, ha, hb):
    l = pl.program_id(0)
    i = pl.program_id(1)

    @pl.when((l == 0) & (i == 0))
    def _():
        ha[...] = h1_full[...]
        hb[...] = h1_full[...].astype(jnp.bfloat16)

    @pl.when((l == 1) & (i == 0))
    def _():
        hb[...] = ha[...].astype(jnp.bfloat16)

    hrow = ha[pl.ds(i * 256, 256), :]
    acc = hrow + jnp.dot(
        bb_blk[...], hb[...], preferred_element_type=jnp.float32
    )
    ha[pl.ds(i * 256, 256), :] = acc
    hout_blk[...] = acc


def _layers23(bb, h1):
    return pl.pallas_call(
        _mm23_body,
        grid=(2, PAD // 256),
        in_specs=[
            pl.BlockSpec((256, PAD), lambda l, i: (i, 0)),
            pl.BlockSpec((PAD, D), lambda l, i: (0, 0)),
        ],
        out_specs=pl.BlockSpec((256, D), lambda l, i: (i, 0)),
        out_shape=jax.ShapeDtypeStruct((PAD, D), jnp.float32),
        scratch_shapes=[
            pltpu.VMEM((PAD, D), jnp.float32),
            pltpu.VMEM((PAD, D), jnp.bfloat16),
        ],
    )(bb, h1)


def kernel(x, edge_index):
    bref = jax.new_ref(jnp.empty((PAD * PAD,), jnp.float32))
    _build_b(edge_index.reshape(-1), bref)
    bmat = bref[...].reshape(PAD, PAD)
    h = jnp.zeros((PAD, D), jnp.float32).at[:N].set(x)
    h, bb = _layer1(bmat, h.astype(jnp.bfloat16), h)
    h = _layers23(bb, h)
    return h[:N]


# speedup vs baseline: 1.5548x; 1.1255x over previous
import jax
import jax.numpy as jnp
from jax import lax
from jax.experimental import pallas as pl
from jax.experimental.pallas import tpu as pltpu
from jax.experimental.pallas import tpu_sc as plsc

N = 10000
E = 160000
D = 256
PAD = 10240
L = 16
EPS = E // 16
GROUPS = EPS // L
KN = 20480
KNG = KN // L
EC = 2000
OUT_KEY = PAD * PAD
HALFK = (PAD // 2) * PAD

SR = 128
SPW = SR * PAD
PASSES = (PAD // 2) // SR
SHARE = SPW // 16
ZN = 2048
CH = 128
CROWS = (KN + CH - 1) // CH


def _build_body(edge_hbm, b_hbm, s_c, d_c, key_v, chunk_v,
                zero_v, ones_v, sp, sem, out_sem):
    c = lax.axis_index("c")
    sidx = lax.axis_index("s")
    base = sidx * EPS

    zero32f = jnp.zeros((L,), jnp.float32)
    onef = jnp.full((L,), 1.0, jnp.float32)

    def zb(g, carry):
        zero_v[pl.ds(g * L, L)] = zero32f
        return carry

    lax.fori_loop(0, ZN // L, zb, 0)
    for g in range(CH // L):
        ones_v[pl.ds(g * L, L)] = onef

    big = jnp.full((L,), OUT_KEY, jnp.int32)

    def bigb(g, carry):
        key_v[pl.ds(g * L, L)] = big
        return carry

    lax.fori_loop(0, KNG, bigb, 0)

    io16 = jnp.arange(L, dtype=jnp.int32)
    one_i = jnp.int32(1)
    zero_i = jnp.int32(0)

    def kb(g, carry):
        cnt0, cnt1 = carry
        sv = s_c[pl.ds(g * L, L)]
        dv = d_c[pl.ds(g * L, L)]
        for kv in (dv * PAD + sv, sv * PAD + dv):
            m0 = kv < HALFK
            t = jnp.where(m0, cnt0 * L, (KNG - 1 - cnt1) * L) + io16
            plsc.store_scatter(key_v, [t], kv)
            cnt0 = cnt0 + jnp.where(m0, one_i, zero_i)
            cnt1 = cnt1 + jnp.where(m0, zero_i, one_i)
        return cnt0, cnt1

    zv = jnp.zeros((L,), jnp.int32)
    carry = (zv, zv)
    for e in range(EPS // EC):
        pltpu.sync_copy(edge_hbm.at[pl.ds(base + e * EC, EC)], s_c)
        pltpu.sync_copy(edge_hbm.at[pl.ds(E + base + e * EC, EC)], d_c)
        carry = lax.fori_loop(0, EC // L, kb, carry)
    cnt0, cnt1 = carry

    def _vmax(v):
        m = v[0]
        for j in range(1, L):
            m = jnp.maximum(m, v[j])
        return m

    g_lo = jnp.where(c == 0, 0, KNG - _vmax(cnt1))
    g_hi = jnp.where(c == 0, _vmax(cnt0), KNG)

    garbv = jnp.full((L,), SPW, jnp.int32)

    def garb_rows(r, carry):
        for l in range(CH // L):
            chunk_v[r, pl.ds(l * L, L)] = garbv
        return carry

    lax.fori_loop(0, CROWS, garb_rows, 0)
    TRASH_ROW = jnp.int32(CROWS)

    my_share = sp.at[pl.ds(sidx * SHARE, SHARE)]

    def pass_body(p, carry):
        lo = (c * (PAD // 2) + p * SR) * PAD
        hi = lo + SPW

        @pl.when(p > 0)
        def _():
            pltpu.make_async_copy(b_hbm.at[pl.ds(0, SHARE)], my_share,
                                  out_sem).wait()

        zcopies = [
            pltpu.async_copy(zero_v,
                             sp.at[pl.ds(sidx * SHARE + z * ZN, ZN)], sem)
            for z in range(SHARE // ZN)
        ]

        def fb(g, cnt_v):
            kv = key_v[pl.ds(g * L, L)]
            m = (kv >= lo) & (kv < hi)
            slot = cnt_v * L + io16
            row = jnp.right_shift(slot, 7)
            col = jnp.bitwise_and(slot, jnp.int32(CH - 1))
            t_row = jnp.where(m, row, TRASH_ROW)
            plsc.store_scatter(chunk_v, [t_row, col], kv - lo)
            return cnt_v + jnp.where(m, one_i, zero_i)

        cnt_v = lax.fori_loop(g_lo, g_hi, fb, zv)
        cmax = _vmax(cnt_v)
        nch = lax.div(cmax * L + (CH - 1), CH)
        for cp in zcopies:
            cp.wait()
        plsc.subcore_barrier()

        def cb(r, carry):
            pltpu.async_copy(ones_v, sp.at[chunk_v.at[r]], sem,
                             add=True).wait()
            return carry

        lax.fori_loop(0, nch, cb, 0)
        plsc.subcore_barrier()

        pltpu.async_copy(my_share, b_hbm.at[pl.ds(lo + sidx * SHARE, SHARE)],
                         out_sem)
        lax.fori_loop(0, nch, garb_rows, 0)
        return carry

    lax.fori_loop(0, PASSES, pass_body, 0)
    pltpu.make_async_copy(b_hbm.at[pl.ds(0, SHARE)], my_share,
                          out_sem).wait()


_build_b = pl.kernel(
    _build_body,
    out_type=(),
    mesh=plsc.VectorSubcoreMesh(core_axis_name="c", subcore_axis_name="s"),
    compiler_params=pltpu.CompilerParams(needs_layout_passes=False),
    scratch_types=[
        pltpu.VMEM((EC,), jnp.int32),
        pltpu.VMEM((EC,), jnp.int32),
        pltpu.VMEM((KN,), jnp.int32),
        pltpu.VMEM((CROWS + 1, CH), jnp.int32),
        pltpu.VMEM((ZN,), jnp.float32),
        pltpu.VMEM((CH,), jnp.float32),
        pltpu.VMEM_SHARED((SPW + L,), jnp.float32),
        pltpu.SemaphoreType.DMA,
        pltpu.SemaphoreType.DMA,
    ],
)


def _mm1_body(b_blk, hb_full, h_blk, o_blk, bb_blk):
    bb = jnp.minimum(b_blk[...], 1.0).astype(jnp.bfloat16)
    bb_blk[...] = bb
    o_blk[...] = h_blk[...] + jnp.dot(
        bb, hb_full[...], preferred_element_type=jnp.float32
    )


def _layer1(bmat, hb, h):
    return pl.pallas_call(
        _mm1_body,
        grid=(PAD // 256,),
        in_specs=[
            pl.BlockSpec((256, PAD), lambda i: (i, 0)),
            pl.BlockSpec((PAD, D), lambda i: (0, 0)),
            pl.BlockSpec((256, D), lambda i: (i, 0)),
        ],
        out_specs=[
            pl.BlockSpec((256, D), lambda i: (i, 0)),
            pl.BlockSpec((256, PAD), lambda i: (i, 0)),
        ],
        out_shape=[
            jax.ShapeDtypeStruct((PAD, D), jnp.float32),
            jax.ShapeDtypeStruct((PAD, PAD), jnp.bfloat16),
        ],
    )(bmat, hb, h)


def _mm23_body(bb_blk, h1_full, hout_blk, ha, hb):
    l = pl.program_id(0)
    i = pl.program_id(1)

    @pl.when((l == 0) & (i == 0))
    def _():
        ha[...] = h1_full[...]
        hb[...] = h1_full[...].astype(jnp.bfloat16)

    @pl.when((l == 1) & (i == 0))
    def _():
        hb[...] = ha[...].astype(jnp.bfloat16)

    hrow = ha[pl.ds(i * 256, 256), :]
    acc = hrow + jnp.dot(
        bb_blk[...], hb[...], preferred_element_type=jnp.float32
    )
    ha[pl.ds(i * 256, 256), :] = acc
    hout_blk[...] = acc


def _layers23(bb, h1):
    return pl.pallas_call(
        _mm23_body,
        grid=(2, PAD // 256),
        in_specs=[
            pl.BlockSpec((256, PAD), lambda l, i: (i, 0)),
            pl.BlockSpec((PAD, D), lambda l, i: (0, 0)),
        ],
        out_specs=pl.BlockSpec((256, D), lambda l, i: (i, 0)),
        out_shape=jax.ShapeDtypeStruct((PAD, D), jnp.float32),
        scratch_shapes=[
            pltpu.VMEM((PAD, D), jnp.float32),
            pltpu.VMEM((PAD, D), jnp.bfloat16),
        ],
    )(bb, h1)


def kernel(x, edge_index):
    bref = jax.new_ref(jnp.empty((PAD * PAD,), jnp.float32))
    _build_b(edge_index.reshape(-1), bref)
    bmat = bref[...].reshape(PAD, PAD)
    h = jnp.zeros((PAD, D), jnp.float32).at[:N].set(x)
    h, bb = _layer1(bmat, h.astype(jnp.bfloat16), h)
    h = _layers23(bb, h)
    return h[:N]
